# baseline (speedup 1.0000x reference)
import functools

import jax
import jax.numpy as jnp
from jax.experimental import pallas as pl
from jax.experimental.pallas import tpu as pltpu

_K = 2048
_PAD = 10240
_RB = 1000
_SIGN = -2147483648
_POS = 2147483647


def _radix_select(u):
    prefix = jnp.int32(0)
    k = jnp.float32(_K)
    for ph in range(8):
        sh = 28 - 4 * ph
        if ph == 0:
            active = jnp.ones(u.shape, dtype=jnp.bool_)
        else:
            active = (u >> (sh + 4)) == (prefix >> (sh + 4))
        digit = (u >> sh) & 15
        b3 = jax.lax.broadcasted_iota(jnp.int32, (16,) + u.shape, 0)
        o3 = ((digit[None] == b3) & active[None]).astype(jnp.float32)
        cnt = jnp.sum(jnp.sum(o3, axis=1), axis=1)
        vv = jax.lax.broadcasted_iota(jnp.int32, (16, 16), 0)
        ww = jax.lax.broadcasted_iota(jnp.int32, (16, 16), 1)
        smat = jnp.where(ww >= vv, cnt[None, :], 0.0)
        suf = jnp.sum(smat, axis=1)
        vstar = jnp.sum((suf >= k).astype(jnp.int32)) - 1
        s_next = jnp.sum(
            jnp.where(jax.lax.iota(jnp.int32, 16) == vstar + 1, suf, 0.0))
        k = k - s_next
        prefix = prefix | (vstar << sh)
    return prefix, k


def _select(g2):
    b = jax.lax.bitcast_convert_type(g2, jnp.int32)
    s = jnp.where(b < 0, b ^ jnp.int32(_POS), b)
    u = s ^ jnp.int32(_SIGN)

    prefix, need = _radix_select(u)
    ts = prefix ^ jnp.int32(_SIGN)

    eqf = (s == ts).astype(jnp.float32)
    ci = jax.lax.broadcasted_iota(jnp.int32, (128, 128), 0)
    cj = jax.lax.broadcasted_iota(jnp.int32, (128, 128), 1)
    slt = jnp.where(ci < cj, 1.0, 0.0)
    lane_excl = jnp.dot(eqf, slt, preferred_element_type=jnp.float32)
    rc = jnp.sum(eqf, axis=1, keepdims=True)
    ri = jax.lax.broadcasted_iota(jnp.int32, (80, 80), 0)
    rj = jax.lax.broadcasted_iota(jnp.int32, (80, 80), 1)
    mrow = jnp.where(rj < ri, 1.0, 0.0)
    row_excl = jnp.dot(mrow, rc, preferred_element_type=jnp.float32)
    p_rank = row_excl + lane_excl
    r_iota = jax.lax.broadcasted_iota(jnp.int32, (80, 128), 0)
    c_iota = jax.lax.broadcasted_iota(jnp.int32, (80, 128), 1)
    idx2 = r_iota * 128 + c_iota
    last_sel = (eqf > 0.0) & (p_rank == need - 1.0)
    m = jnp.sum(jnp.where(last_sel, idx2, 0))
    return ts, m


def _body(g2_ref, x_ref, w_ref, b_ref, g3_ref, o_ref, tsm_ref, *, scale):
    i = pl.program_id(0)

    @pl.when(i == 0)
    def _():
        ts, m = _select(g2_ref[...])
        tsm_ref[0] = ts
        tsm_ref[1] = m

    @pl.when(i > 0)
    def _():
        ts = tsm_ref[0]
        m = tsm_ref[1]
        gr = g3_ref[...].reshape(1, _RB)
        br = jax.lax.bitcast_convert_type(gr, jnp.int32)
        sr = jnp.where(br < 0, br ^ jnp.int32(_POS), br)
        base = (i - 1) * _RB
        idxr = base + jax.lax.broadcasted_iota(jnp.int32, (1, _RB), 1)
        selr = ((sr > ts) | ((sr == ts) & (idxr <= m))).astype(jnp.float32)
        selc = jnp.transpose(selr, (1, 0))
        y = jnp.dot(x_ref[...], w_ref[...], preferred_element_type=jnp.float32)
        y = (y + b_ref[...]) * scale
        o_ref[...] = y * selc


def kernel(x, edge_index, importance_scores, weight, bias):
    del edge_index
    num_nodes = x.shape[0]
    out_dim = weight.shape[1]
    p = importance_scores / jnp.sum(importance_scores)
    g = jax.random.gumbel(jax.random.key(42), (num_nodes,), jnp.float32)
    g = g + jnp.log(p)
    g_pad = jnp.concatenate(
        [g, jnp.full((_PAD - num_nodes,), -jnp.inf, dtype=jnp.float32)])
    g2 = g_pad.reshape(80, 128)
    nblk = num_nodes // _RB
    g3 = g.reshape(nblk, 1, _RB)

    scale = num_nodes / _K

    def mm_idx(i):
        j = jnp.maximum(i - 1, 0)
        return (j, 0)

    def mm_idx3(i):
        j = jnp.maximum(i - 1, 0)
        return (j, 0, 0)

    out = pl.pallas_call(
        functools.partial(_body, scale=scale),
        grid=(nblk + 1,),
        in_specs=[
            pl.BlockSpec((80, 128), lambda i: (0, 0)),
            pl.BlockSpec((_RB, x.shape[1]), mm_idx),
            pl.BlockSpec((x.shape[1], out_dim), lambda i: (0, 0)),
            pl.BlockSpec((1, out_dim), lambda i: (0, 0)),
            pl.BlockSpec((1, 1, _RB), mm_idx3),
        ],
        out_specs=pl.BlockSpec((_RB, out_dim), mm_idx),
        out_shape=jax.ShapeDtypeStruct((num_nodes, out_dim), jnp.float32),
        scratch_shapes=[pltpu.SMEM((2,), jnp.int32)],
    )(g2, x, weight, bias.reshape(1, out_dim), g3)
    return out


# speedup vs baseline: 1.0181x; 1.0181x over previous
import functools

import jax
import jax.numpy as jnp
from jax.experimental import pallas as pl
from jax.experimental.pallas import tpu as pltpu

_K = 2048
_PAD = 10240
_RB = 1000
_SIGN = -2147483648
_POS = 2147483647


def _radix_select(u):
    prefix = jnp.int32(0)
    k = jnp.float32(_K)
    for ph in range(8):
        sh = 28 - 4 * ph
        if ph == 0:
            active = jnp.ones(u.shape, dtype=jnp.bool_)
        else:
            active = (u >> (sh + 4)) == (prefix >> (sh + 4))
        digit = (u >> sh) & 15
        b3 = jax.lax.broadcasted_iota(jnp.int32, (16,) + u.shape, 0)
        o3 = ((digit[None] == b3) & active[None]).astype(jnp.float32)
        cnt = jnp.sum(jnp.sum(o3, axis=1), axis=1)
        vv = jax.lax.broadcasted_iota(jnp.int32, (16, 16), 0)
        ww = jax.lax.broadcasted_iota(jnp.int32, (16, 16), 1)
        smat = jnp.where(ww >= vv, cnt[None, :], 0.0)
        suf = jnp.sum(smat, axis=1)
        vstar = jnp.sum((suf >= k).astype(jnp.int32)) - 1
        s_next = jnp.sum(
            jnp.where(jax.lax.iota(jnp.int32, 16) == vstar + 1, suf, 0.0))
        k = k - s_next
        prefix = prefix | (vstar << sh)
    return prefix, k


def _select(g2):
    b = jax.lax.bitcast_convert_type(g2, jnp.int32)
    s = jnp.where(b < 0, b ^ jnp.int32(_POS), b)
    u = s ^ jnp.int32(_SIGN)

    prefix, need = _radix_select(u)
    ts = prefix ^ jnp.int32(_SIGN)

    eqf = (s == ts).astype(jnp.float32)
    ci = jax.lax.broadcasted_iota(jnp.int32, (128, 128), 0)
    cj = jax.lax.broadcasted_iota(jnp.int32, (128, 128), 1)
    slt = jnp.where(ci < cj, 1.0, 0.0)
    lane_excl = jnp.dot(eqf, slt, preferred_element_type=jnp.float32)
    rc = jnp.sum(eqf, axis=1, keepdims=True)
    ri = jax.lax.broadcasted_iota(jnp.int32, (80, 80), 0)
    rj = jax.lax.broadcasted_iota(jnp.int32, (80, 80), 1)
    mrow = jnp.where(rj < ri, 1.0, 0.0)
    row_excl = jnp.dot(mrow, rc, preferred_element_type=jnp.float32)
    p_rank = row_excl + lane_excl
    r_iota = jax.lax.broadcasted_iota(jnp.int32, (80, 128), 0)
    c_iota = jax.lax.broadcasted_iota(jnp.int32, (80, 128), 1)
    idx2 = r_iota * 128 + c_iota
    last_sel = (eqf > 0.0) & (p_rank == need - 1.0)
    m = jnp.sum(jnp.where(last_sel, idx2, 0))
    return ts, m


def _body(g2_ref, x_ref, w_ref, b_ref, g3_ref, o_ref, tsm_ref, *, scale):
    i = pl.program_id(0)

    @pl.when(i == 0)
    def _():
        ts, m = _select(g2_ref[...])
        tsm_ref[0] = ts
        tsm_ref[1] = m

    ts = tsm_ref[0]
    m = tsm_ref[1]
    gr = g3_ref[...].reshape(1, _RB)
    br = jax.lax.bitcast_convert_type(gr, jnp.int32)
    sr = jnp.where(br < 0, br ^ jnp.int32(_POS), br)
    base = i * _RB
    idxr = base + jax.lax.broadcasted_iota(jnp.int32, (1, _RB), 1)
    selr = ((sr > ts) | ((sr == ts) & (idxr <= m))).astype(jnp.float32)
    selc = jnp.transpose(selr, (1, 0))
    y = jnp.dot(x_ref[...], w_ref[...], preferred_element_type=jnp.float32)
    y = (y + b_ref[...]) * scale
    o_ref[...] = y * selc


def kernel(x, edge_index, importance_scores, weight, bias):
    del edge_index
    num_nodes = x.shape[0]
    out_dim = weight.shape[1]
    p = importance_scores / jnp.sum(importance_scores)
    g = jax.random.gumbel(jax.random.key(42), (num_nodes,), jnp.float32)
    g = g + jnp.log(p)
    g_pad = jnp.concatenate(
        [g, jnp.full((_PAD - num_nodes,), -jnp.inf, dtype=jnp.float32)])
    g2 = g_pad.reshape(80, 128)
    nblk = num_nodes // _RB
    g3 = g.reshape(nblk, 1, _RB)

    scale = num_nodes / _K

    def mm_idx(i):
        return (i, 0)

    def mm_idx3(i):
        return (i, 0, 0)

    out = pl.pallas_call(
        functools.partial(_body, scale=scale),
        grid=(nblk,),
        in_specs=[
            pl.BlockSpec((80, 128), lambda i: (0, 0)),
            pl.BlockSpec((_RB, x.shape[1]), mm_idx),
            pl.BlockSpec((x.shape[1], out_dim), lambda i: (0, 0)),
            pl.BlockSpec((1, out_dim), lambda i: (0, 0)),
            pl.BlockSpec((1, 1, _RB), mm_idx3),
        ],
        out_specs=pl.BlockSpec((_RB, out_dim), mm_idx),
        out_shape=jax.ShapeDtypeStruct((num_nodes, out_dim), jnp.float32),
        scratch_shapes=[pltpu.SMEM((2,), jnp.int32)],
    )(g2, x, weight, bias.reshape(1, out_dim), g3)
    return out


# speedup vs baseline: 1.1565x; 1.1360x over previous
import functools

import jax
import jax.numpy as jnp
from jax.experimental import pallas as pl
from jax.experimental.pallas import tpu as pltpu

_K = 2048
_PAD = 10240
_RB = 2000
_GW = 1000
_SIGN = -2147483648
_POS = 2147483647


def _radix_select(u):
    prefix = jnp.int32(0)
    k = jnp.float32(_K)
    for ph in range(8):
        sh = 28 - 4 * ph
        if ph == 0:
            active = jnp.ones(u.shape, dtype=jnp.bool_)
        else:
            active = (u >> (sh + 4)) == (prefix >> (sh + 4))
        digit = (u >> sh) & 15
        b3 = jax.lax.broadcasted_iota(jnp.int32, (16,) + u.shape, 0)
        o3 = ((digit[None] == b3) & active[None]).astype(jnp.float32)
        cnt = jnp.sum(jnp.sum(o3, axis=1), axis=1)
        vv = jax.lax.broadcasted_iota(jnp.int32, (16, 16), 0)
        ww = jax.lax.broadcasted_iota(jnp.int32, (16, 16), 1)
        smat = jnp.where(ww >= vv, cnt[None, :], 0.0)
        suf = jnp.sum(smat, axis=1)
        vstar = jnp.sum((suf >= k).astype(jnp.int32)) - 1
        s_next = jnp.sum(
            jnp.where(jax.lax.iota(jnp.int32, 16) == vstar + 1, suf, 0.0))
        k = k - s_next
        prefix = prefix | (vstar << sh)
    return prefix, k


def _select(g2):
    b = jax.lax.bitcast_convert_type(g2, jnp.int32)
    s = jnp.where(b < 0, b ^ jnp.int32(_POS), b)
    u = s ^ jnp.int32(_SIGN)

    prefix, need = _radix_select(u)
    ts = prefix ^ jnp.int32(_SIGN)

    eqf = (s == ts).astype(jnp.float32)
    ci = jax.lax.broadcasted_iota(jnp.int32, (128, 128), 0)
    cj = jax.lax.broadcasted_iota(jnp.int32, (128, 128), 1)
    slt = jnp.where(ci < cj, 1.0, 0.0)
    lane_excl = jnp.dot(eqf, slt, preferred_element_type=jnp.float32)
    rc = jnp.sum(eqf, axis=1, keepdims=True)
    ri = jax.lax.broadcasted_iota(jnp.int32, (80, 80), 0)
    rj = jax.lax.broadcasted_iota(jnp.int32, (80, 80), 1)
    mrow = jnp.where(rj < ri, 1.0, 0.0)
    row_excl = jnp.dot(mrow, rc, preferred_element_type=jnp.float32)
    p_rank = row_excl + lane_excl
    r_iota = jax.lax.broadcasted_iota(jnp.int32, (80, 128), 0)
    c_iota = jax.lax.broadcasted_iota(jnp.int32, (80, 128), 1)
    idx2 = r_iota * 128 + c_iota
    last_sel = (eqf > 0.0) & (p_rank == need - 1.0)
    m = jnp.sum(jnp.where(last_sel, idx2, 0))
    return ts, m


def _body(g2_ref, x_ref, w_ref, b_ref, g3_ref, o_ref, tsm_ref, *, scale):
    i = pl.program_id(0)

    @pl.when(i == 0)
    def _():
        ts, m = _select(g2_ref[...])
        tsm_ref[0] = ts
        tsm_ref[1] = m

    ts = tsm_ref[0]
    m = tsm_ref[1]
    cols = []
    for j in range(_RB // _GW):
        gr = g3_ref[j].reshape(1, _GW)
        br = jax.lax.bitcast_convert_type(gr, jnp.int32)
        sr = jnp.where(br < 0, br ^ jnp.int32(_POS), br)
        base = i * _RB + j * _GW
        idxr = base + jax.lax.broadcasted_iota(jnp.int32, (1, _GW), 1)
        selr = ((sr > ts) | ((sr == ts) & (idxr <= m))).astype(jnp.float32)
        cols.append(jnp.transpose(selr, (1, 0)))
    selc = jnp.concatenate(cols, axis=0)
    y = jnp.dot(x_ref[...], w_ref[...], preferred_element_type=jnp.float32)
    y = (y + b_ref[...]) * scale
    o_ref[...] = y * selc


def kernel(x, edge_index, importance_scores, weight, bias):
    del edge_index
    num_nodes = x.shape[0]
    out_dim = weight.shape[1]
    p = importance_scores / jnp.sum(importance_scores)
    g = jax.random.gumbel(jax.random.key(42), (num_nodes,), jnp.float32)
    g = g + jnp.log(p)
    g_pad = jnp.concatenate(
        [g, jnp.full((_PAD - num_nodes,), -jnp.inf, dtype=jnp.float32)])
    g2 = g_pad.reshape(80, 128)
    nblk = num_nodes // _RB
    g3 = g.reshape(num_nodes // _GW, 1, _GW)

    scale = num_nodes / _K

    def mm_idx(i):
        return (i, 0)

    def mm_idx3(i):
        return (i, 0, 0)

    out = pl.pallas_call(
        functools.partial(_body, scale=scale),
        grid=(nblk,),
        in_specs=[
            pl.BlockSpec((80, 128), lambda i: (0, 0)),
            pl.BlockSpec((_RB, x.shape[1]), mm_idx),
            pl.BlockSpec((x.shape[1], out_dim), lambda i: (0, 0)),
            pl.BlockSpec((1, out_dim), lambda i: (0, 0)),
            pl.BlockSpec((_RB // _GW, 1, _GW), mm_idx3),
        ],
        out_specs=pl.BlockSpec((_RB, out_dim), mm_idx),
        out_shape=jax.ShapeDtypeStruct((num_nodes, out_dim), jnp.float32),
        scratch_shapes=[pltpu.SMEM((2,), jnp.int32)],
    )(g2, x, weight, bias.reshape(1, out_dim), g3)
    return out


# speedup vs baseline: 1.3510x; 1.1682x over previous
import functools

import jax
import jax.numpy as jnp
from jax.experimental import pallas as pl
from jax.experimental.pallas import tpu as pltpu

_K = 2048
_PAD = 10240
_RB = 5000
_GW = 1000
_SIGN = -2147483648
_POS = 2147483647


def _radix_select(u):
    prefix = jnp.int32(0)
    k = jnp.float32(_K)
    for ph in range(8):
        sh = 28 - 4 * ph
        if ph == 0:
            active = jnp.ones(u.shape, dtype=jnp.bool_)
        else:
            active = (u >> (sh + 4)) == (prefix >> (sh + 4))
        digit = (u >> sh) & 15
        b3 = jax.lax.broadcasted_iota(jnp.int32, (16,) + u.shape, 0)
        o3 = ((digit[None] == b3) & active[None]).astype(jnp.float32)
        cnt = jnp.sum(jnp.sum(o3, axis=1), axis=1)
        vv = jax.lax.broadcasted_iota(jnp.int32, (16, 16), 0)
        ww = jax.lax.broadcasted_iota(jnp.int32, (16, 16), 1)
        smat = jnp.where(ww >= vv, cnt[None, :], 0.0)
        suf = jnp.sum(smat, axis=1)
        vstar = jnp.sum((suf >= k).astype(jnp.int32)) - 1
        s_next = jnp.sum(
            jnp.where(jax.lax.iota(jnp.int32, 16) == vstar + 1, suf, 0.0))
        k = k - s_next
        prefix = prefix | (vstar << sh)
    return prefix, k


def _select(g2):
    b = jax.lax.bitcast_convert_type(g2, jnp.int32)
    s = jnp.where(b < 0, b ^ jnp.int32(_POS), b)
    u = s ^ jnp.int32(_SIGN)

    prefix, need = _radix_select(u)
    ts = prefix ^ jnp.int32(_SIGN)

    eqf = (s == ts).astype(jnp.float32)
    ci = jax.lax.broadcasted_iota(jnp.int32, (128, 128), 0)
    cj = jax.lax.broadcasted_iota(jnp.int32, (128, 128), 1)
    slt = jnp.where(ci < cj, 1.0, 0.0)
    lane_excl = jnp.dot(eqf, slt, preferred_element_type=jnp.float32)
    rc = jnp.sum(eqf, axis=1, keepdims=True)
    ri = jax.lax.broadcasted_iota(jnp.int32, (80, 80), 0)
    rj = jax.lax.broadcasted_iota(jnp.int32, (80, 80), 1)
    mrow = jnp.where(rj < ri, 1.0, 0.0)
    row_excl = jnp.dot(mrow, rc, preferred_element_type=jnp.float32)
    p_rank = row_excl + lane_excl
    r_iota = jax.lax.broadcasted_iota(jnp.int32, (80, 128), 0)
    c_iota = jax.lax.broadcasted_iota(jnp.int32, (80, 128), 1)
    idx2 = r_iota * 128 + c_iota
    last_sel = (eqf > 0.0) & (p_rank == need - 1.0)
    m = jnp.sum(jnp.where(last_sel, idx2, 0))
    return ts, m


def _body(g2_ref, x_ref, w_ref, b_ref, g3_ref, o_ref, tsm_ref, *, scale):
    i = pl.program_id(0)

    @pl.when(i == 0)
    def _():
        ts, m = _select(g2_ref[...])
        tsm_ref[0] = ts
        tsm_ref[1] = m

    ts = tsm_ref[0]
    m = tsm_ref[1]
    cols = []
    for j in range(_RB // _GW):
        gr = g3_ref[j].reshape(1, _GW)
        br = jax.lax.bitcast_convert_type(gr, jnp.int32)
        sr = jnp.where(br < 0, br ^ jnp.int32(_POS), br)
        base = i * _RB + j * _GW
        idxr = base + jax.lax.broadcasted_iota(jnp.int32, (1, _GW), 1)
        selr = ((sr > ts) | ((sr == ts) & (idxr <= m))).astype(jnp.float32)
        cols.append(jnp.transpose(selr, (1, 0)))
    selc = jnp.concatenate(cols, axis=0)
    y = jnp.dot(x_ref[...], w_ref[...], preferred_element_type=jnp.float32)
    y = (y + b_ref[...]) * scale
    o_ref[...] = y * selc


def kernel(x, edge_index, importance_scores, weight, bias):
    del edge_index
    num_nodes = x.shape[0]
    out_dim = weight.shape[1]
    p = importance_scores / jnp.sum(importance_scores)
    g = jax.random.gumbel(jax.random.key(42), (num_nodes,), jnp.float32)
    g = g + jnp.log(p)
    g_pad = jnp.concatenate(
        [g, jnp.full((_PAD - num_nodes,), -jnp.inf, dtype=jnp.float32)])
    g2 = g_pad.reshape(80, 128)
    nblk = num_nodes // _RB
    g3 = g.reshape(num_nodes // _GW, 1, _GW)

    scale = num_nodes / _K

    def mm_idx(i):
        return (i, 0)

    def mm_idx3(i):
        return (i, 0, 0)

    out = pl.pallas_call(
        functools.partial(_body, scale=scale),
        grid=(nblk,),
        in_specs=[
            pl.BlockSpec((80, 128), lambda i: (0, 0)),
            pl.BlockSpec((_RB, x.shape[1]), mm_idx),
            pl.BlockSpec((x.shape[1], out_dim), lambda i: (0, 0)),
            pl.BlockSpec((1, out_dim), lambda i: (0, 0)),
            pl.BlockSpec((_RB // _GW, 1, _GW), mm_idx3),
        ],
        out_specs=pl.BlockSpec((_RB, out_dim), mm_idx),
        out_shape=jax.ShapeDtypeStruct((num_nodes, out_dim), jnp.float32),
        scratch_shapes=[pltpu.SMEM((2,), jnp.int32)],
    )(g2, x, weight, bias.reshape(1, out_dim), g3)
    return out
